# baseline (speedup 1.0000x reference)
import functools

import jax
import jax.numpy as jnp
from jax import lax
from jax.experimental import pallas as pl
from jax.experimental.pallas import tpu as pltpu
from jax.experimental.pallas import tpu_sc as plsc

N = 10000
E = 320000
D = 128
H = 5
CLS = 16

NC = 2
NS = 16
NW = NC * NS
CH = 128
R = 80
EPAD = NW * CH * R
NP = 10240
GF = NP * H

_mesh = plsc.VectorSubcoreMesh(core_axis_name="c", subcore_axis_name="s")


@functools.partial(
    pl.kernel,
    out_type=jax.ShapeDtypeStruct((NW, NP), jnp.float32),
    mesh=_mesh,
    compiler_params=pltpu.CompilerParams(needs_layout_passes=False),
    scratch_types=[
        pltpu.VMEM((R, CH), jnp.int32),
        pltpu.VMEM((NP,), jnp.float32),
    ],
)
def _sc_deg(dst_hbm, z_hbm, out_hbm, dstv, accv):
    c = lax.axis_index("c")
    s = lax.axis_index("s")
    b = c * NS + s

    pltpu.sync_copy(z_hbm, accv)
    pltpu.sync_copy(dst_hbm.at[b], dstv)
    ones = jnp.ones((16,), jnp.float32)

    def step(j, carry):
        for t in range(CH // 16):
            d0 = dstv[j, pl.ds(t * 16, 16)]
            plsc.addupdate_scatter(accv, [d0], ones)
        return carry

    lax.fori_loop(0, R, step, 0)
    pltpu.sync_copy(accv, out_hbm.at[b])


@functools.partial(
    pl.kernel,
    out_type=jax.ShapeDtypeStruct((NW, GF), jnp.float32),
    mesh=_mesh,
    compiler_params=pltpu.CompilerParams(needs_layout_passes=False),
    scratch_types=[
        pltpu.VMEM((R, CH), jnp.int32),
        pltpu.VMEM((R, CH), jnp.int32),
        pltpu.VMEM((GF,), jnp.float32),
        pltpu.VMEM((GF,), jnp.float32),
    ],
)
def _sc_pass(src_hbm, dst_hbm, g_hbm, z_hbm, out_hbm, srcv, dstv, gv, accv):
    c = lax.axis_index("c")
    s = lax.axis_index("s")
    b = c * NS + s

    pltpu.sync_copy(z_hbm, accv)
    pltpu.sync_copy(g_hbm, gv)
    pltpu.sync_copy(src_hbm.at[b], srcv)
    pltpu.sync_copy(dst_hbm.at[b], dstv)

    def step(j, carry):
        for t in range(CH // 16):
            s0 = srcv[j, pl.ds(t * 16, 16)]
            d0 = dstv[j, pl.ds(t * 16, 16)]
            for k in range(H):
                vals = plsc.load_gather(gv, [s0 + k * NP])
                plsc.addupdate_scatter(accv, [d0 + k * NP], vals)
        return carry

    lax.fori_loop(0, R, step, 0)
    pltpu.sync_copy(accv, out_hbm.at[b])


def _tc_prep_body(x_ref, w_ref, degw_ref, g1_ref, dr_ref):
    deg = jnp.ones((1, NP), jnp.float32)
    for w in range(NW):
        deg = deg + degw_ref[w : w + 1, :]
    dinv = lax.rsqrt(deg)
    ht = lax.dot_general(w_ref[...], x_ref[...], (((0,), (1,)), ((), ())),
                         preferred_element_type=jnp.float32)
    g1_ref[...] = ht * dinv
    dr_ref[...] = jnp.broadcast_to(dinv, (H, NP))


def _tc_mid_body(aw_ref, g1_ref, dr_ref, b1_ref, g2_ref):
    a = g1_ref[...]
    for w in range(NW):
        a = a + aw_ref[w]
    out1 = dr_ref[...] * a + b1_ref[...]
    r = jnp.maximum(out1, 0.0)
    col = lax.broadcasted_iota(jnp.int32, (H, NP), 1)
    r = jnp.where(col < N, r, 0.0)
    g2_ref[...] = dr_ref[...] * r


def _tc_final_body(aw_ref, g2_ref, dr_ref, w2_ref, b2_ref, o_ref):
    a = g2_ref[...]
    for w in range(NW):
        a = a + aw_ref[w]
    u = dr_ref[...] * a
    logits = lax.dot_general(u, w2_ref[...], (((0,), (0,)), ((), ())),
                             preferred_element_type=jnp.float32) + b2_ref[...]
    m = jnp.max(logits, axis=1, keepdims=True)
    lse = jnp.log(jnp.sum(jnp.exp(logits - m), axis=1, keepdims=True))
    o_ref[...] = logits - m - lse


_tc_prep = pl.pallas_call(
    _tc_prep_body,
    out_shape=[
        jax.ShapeDtypeStruct((H, NP), jnp.float32),
        jax.ShapeDtypeStruct((H, NP), jnp.float32),
    ],
)

_tc_mid = pl.pallas_call(
    _tc_mid_body,
    out_shape=jax.ShapeDtypeStruct((H, NP), jnp.float32),
)

_tc_final = pl.pallas_call(
    _tc_final_body,
    out_shape=jax.ShapeDtypeStruct((NP, CLS), jnp.float32),
)


def kernel(x, edge_index, W1, b1, W2, b2):
    ei = edge_index.astype(jnp.int32)

    padd = jnp.full((EPAD - E,), N, jnp.int32)
    srcd = jnp.concatenate([ei[0], padd]).reshape(NW, R, CH)
    dstd = jnp.concatenate([ei[1], padd]).reshape(NW, R, CH)

    xp = jnp.pad(x.astype(jnp.float32), ((0, NP - N), (0, 0)))
    w1 = W1.astype(jnp.float32)
    b1c = b1.astype(jnp.float32).reshape(H, 1)
    w2 = W2.astype(jnp.float32)
    b2r = b2.astype(jnp.float32).reshape(1, CLS)
    z1 = jnp.zeros((NP,), jnp.float32)
    z5 = jnp.zeros((GF,), jnp.float32)

    degw = _sc_deg(dstd, z1)
    g1, dr = _tc_prep(xp, w1, degw)
    a1 = _sc_pass(srcd, dstd, g1.reshape(-1), z5).reshape(NW, H, NP)
    g2 = _tc_mid(a1, g1, dr, b1c)
    a2 = _sc_pass(srcd, dstd, g2.reshape(-1), z5).reshape(NW, H, NP)
    out = _tc_final(a2, g2, dr, w2, b2r)
    return out[:N]


# speedup vs baseline: 1.2502x; 1.2502x over previous
import functools

import jax
import jax.numpy as jnp
from jax import lax
from jax.experimental import pallas as pl
from jax.experimental.pallas import tpu as pltpu
from jax.experimental.pallas import tpu_sc as plsc

N = 10000
E = 320000
D = 128
H = 5
CLS = 16

NC = 2
NS = 16
NW = NC * NS
CH = 128

R = 80
EPAD = NW * CH * R
NP = 10240
SL = NP // NS

GF = NP * H
GSL = GF // NS

_mesh = plsc.VectorSubcoreMesh(core_axis_name="c", subcore_axis_name="s")


@functools.partial(
    pl.kernel,
    out_type=jax.ShapeDtypeStruct((NC, NP), jnp.float32),
    mesh=_mesh,
    scratch_types=[
        pltpu.VMEM((R, CH), jnp.int32),
        pltpu.VMEM((CH,), jnp.float32),
        pltpu.VMEM_SHARED((NP,), jnp.float32),
    ],
)
def _sc_deg(dst_hbm, z_hbm, out_hbm, dstv, ones_v, acc_sh):
    c = lax.axis_index("c")
    s = lax.axis_index("s")
    b = c * NS + s

    pltpu.sync_copy(z_hbm.at[pl.ds(s * SL, SL)], acc_sh.at[pl.ds(s * SL, SL)])
    pltpu.sync_copy(dst_hbm.at[b], dstv)
    for i in range(CH // 16):
        ones_v[pl.ds(i * 16, 16)] = jnp.ones((16,), jnp.float32)
    plsc.subcore_barrier()

    def step(j, carry):
        pltpu.sync_copy(ones_v, acc_sh.at[dstv.at[j]], add=True)
        return carry

    lax.fori_loop(0, R, step, 0)
    plsc.subcore_barrier()
    pltpu.sync_copy(acc_sh.at[pl.ds(s * SL, SL)], out_hbm.at[c, pl.ds(s * SL, SL)])


@functools.partial(
    pl.kernel,
    out_type=jax.ShapeDtypeStruct((NC, GF), jnp.float32),
    mesh=_mesh,
    compiler_params=pltpu.CompilerParams(needs_layout_passes=False),
    scratch_types=[
        pltpu.VMEM((R, CH), jnp.int32),
        pltpu.VMEM((R, CH), jnp.int32),
        pltpu.VMEM((GF,), jnp.float32),
        pltpu.VMEM((H, CH), jnp.float32),
        pltpu.VMEM((H, CH), jnp.float32),
        pltpu.VMEM((H, CH), jnp.int32),
        pltpu.VMEM((H, CH), jnp.int32),
        pltpu.VMEM_SHARED((GF,), jnp.float32),
        pltpu.SemaphoreType.DMA,
        pltpu.SemaphoreType.DMA,
    ],
)
def _sc_pass(src_hbm, dst_hbm, g_hbm, z_hbm, out_hbm,
             srcv, dstv, gv, rows0, rows1, didx0, didx1, acc_sh, sem0, sem1):
    c = lax.axis_index("c")
    s = lax.axis_index("s")
    b = c * NS + s
    bufs = ((rows0, didx0, sem0), (rows1, didx1, sem1))

    pltpu.sync_copy(z_hbm.at[pl.ds(s * GSL, GSL)], acc_sh.at[pl.ds(s * GSL, GSL)])
    pltpu.sync_copy(g_hbm, gv)
    pltpu.sync_copy(src_hbm.at[b], srcv)
    pltpu.sync_copy(dst_hbm.at[b], dstv)
    plsc.subcore_barrier()

    def gather_and_fire(j, p):
        rows, didx, sem = bufs[p]
        for t in range(CH // 16):
            s0 = srcv[j, pl.ds(t * 16, 16)]
            d0 = dstv[j, pl.ds(t * 16, 16)]
            for k in range(H):
                rows[k, pl.ds(t * 16, 16)] = plsc.load_gather(gv, [s0 + k * NP])
                didx[k, pl.ds(t * 16, 16)] = d0 + k * NP
        for k in range(H):
            pltpu.async_copy(rows.at[k], acc_sh.at[didx.at[k]], sem, add=True)

    def drain(p):
        rows, _, sem = bufs[p]
        for k in range(H):
            pltpu.make_async_copy(z_hbm.at[pl.ds(0, CH)], rows.at[k], sem).wait()

    gather_and_fire(0, 0)
    gather_and_fire(1, 1)

    def step(jj, carry):
        for p in range(2):
            drain(p)
            gather_and_fire(jj * 2 + 2 + p, p)
        return carry

    lax.fori_loop(0, (R - 2) // 2, step, 0)
    drain(0)
    drain(1)

    plsc.subcore_barrier()
    pltpu.sync_copy(acc_sh.at[pl.ds(s * GSL, GSL)], out_hbm.at[c, pl.ds(s * GSL, GSL)])


def _tc_prep_body(x_ref, w_ref, deg2_ref, g1_ref, dr_ref):
    deg = deg2_ref[0:1, :] + deg2_ref[1:2, :] + 1.0
    dinv = lax.rsqrt(deg)
    ht = lax.dot_general(w_ref[...], x_ref[...], (((0,), (1,)), ((), ())),
                         preferred_element_type=jnp.float32)
    g1_ref[...] = ht * dinv
    dr_ref[...] = jnp.broadcast_to(dinv, (H, NP))


def _tc_mid_body(a0_ref, a1_ref, g1_ref, dr_ref, b1_ref, g2_ref):
    out1 = dr_ref[...] * (a0_ref[...] + a1_ref[...] + g1_ref[...]) + b1_ref[...]
    r = jnp.maximum(out1, 0.0)
    col = lax.broadcasted_iota(jnp.int32, (H, NP), 1)
    r = jnp.where(col < N, r, 0.0)
    g2_ref[...] = dr_ref[...] * r


def _tc_final_body(a0_ref, a1_ref, g2_ref, dr_ref, w2_ref, b2_ref, o_ref):
    u = dr_ref[...] * (a0_ref[...] + a1_ref[...] + g2_ref[...])
    logits = lax.dot_general(u, w2_ref[...], (((0,), (0,)), ((), ())),
                             preferred_element_type=jnp.float32) + b2_ref[...]
    m = jnp.max(logits, axis=1, keepdims=True)
    lse = jnp.log(jnp.sum(jnp.exp(logits - m), axis=1, keepdims=True))
    o_ref[...] = logits - m - lse


_tc_prep = pl.pallas_call(
    _tc_prep_body,
    out_shape=[
        jax.ShapeDtypeStruct((H, NP), jnp.float32),
        jax.ShapeDtypeStruct((H, NP), jnp.float32),
    ],
)

_tc_mid = pl.pallas_call(
    _tc_mid_body,
    out_shape=jax.ShapeDtypeStruct((H, NP), jnp.float32),
)

_tc_final = pl.pallas_call(
    _tc_final_body,
    out_shape=jax.ShapeDtypeStruct((NP, CLS), jnp.float32),
)


def kernel(x, edge_index, W1, b1, W2, b2):
    ei = edge_index.astype(jnp.int32)

    padd = jnp.full((EPAD - E,), N, jnp.int32)
    srcd = jnp.concatenate([ei[0], padd]).reshape(NW, R, CH)
    dstd = jnp.concatenate([ei[1], padd]).reshape(NW, R, CH)

    xp = jnp.pad(x.astype(jnp.float32), ((0, NP - N), (0, 0)))
    w1 = W1.astype(jnp.float32)
    b1c = b1.astype(jnp.float32).reshape(H, 1)
    w2 = W2.astype(jnp.float32)
    b2r = b2.astype(jnp.float32).reshape(1, CLS)
    z1 = jnp.zeros((NP,), jnp.float32)
    z5 = jnp.zeros((GF,), jnp.float32)

    degp = _sc_deg(dstd, z1)
    g1, dr = _tc_prep(xp, w1, degp)
    a1 = _sc_pass(srcd, dstd, g1.reshape(-1), z5).reshape(NC, H, NP)
    g2 = _tc_mid(a1[0], a1[1], g1, dr, b1c)
    a2 = _sc_pass(srcd, dstd, g2.reshape(-1), z5).reshape(NC, H, NP)
    out = _tc_final(a2[0], a2[1], g2, dr, w2, b2r)
    return out[:N]


# speedup vs baseline: 1.3020x; 1.0414x over previous
import functools

import jax
import jax.numpy as jnp
from jax import lax
from jax.experimental import pallas as pl
from jax.experimental.pallas import tpu as pltpu
from jax.experimental.pallas import tpu_sc as plsc

N = 10000
E = 320000
D = 128
H = 5
CLS = 16

NC = 2
NS = 16
NW = NC * NS
CH = 128

R = 80
EPAD = NW * CH * R
NP = 10240
SL = NP // NS

GF = NP * H
HS = 3
HT = H - HS
GS = NP * HS
GT = NP * HT
GSL = GS // NS

_mesh = plsc.VectorSubcoreMesh(core_axis_name="c", subcore_axis_name="s")


@functools.partial(
    pl.kernel,
    out_type=jax.ShapeDtypeStruct((NC, NP), jnp.float32),
    mesh=_mesh,
    scratch_types=[
        pltpu.VMEM((R, CH), jnp.int32),
        pltpu.VMEM((CH,), jnp.float32),
        pltpu.VMEM_SHARED((NP,), jnp.float32),
    ],
)
def _sc_deg(dst_hbm, z_hbm, out_hbm, dstv, ones_v, acc_sh):
    c = lax.axis_index("c")
    s = lax.axis_index("s")
    b = c * NS + s

    pltpu.sync_copy(z_hbm.at[pl.ds(s * SL, SL)], acc_sh.at[pl.ds(s * SL, SL)])
    pltpu.sync_copy(dst_hbm.at[b], dstv)
    for i in range(CH // 16):
        ones_v[pl.ds(i * 16, 16)] = jnp.ones((16,), jnp.float32)
    plsc.subcore_barrier()

    def step(j, carry):
        pltpu.sync_copy(ones_v, acc_sh.at[dstv.at[j]], add=True)
        return carry

    lax.fori_loop(0, R, step, 0)
    plsc.subcore_barrier()
    pltpu.sync_copy(acc_sh.at[pl.ds(s * SL, SL)], out_hbm.at[c, pl.ds(s * SL, SL)])


@functools.partial(
    pl.kernel,
    out_type=[
        jax.ShapeDtypeStruct((NC, GS), jnp.float32),
        jax.ShapeDtypeStruct((NW, GT), jnp.float32),
    ],
    mesh=_mesh,
    compiler_params=pltpu.CompilerParams(needs_layout_passes=False),
    scratch_types=[
        pltpu.VMEM((R, CH), jnp.int32),
        pltpu.VMEM((R, CH), jnp.int32),
        pltpu.VMEM((GF,), jnp.float32),
        pltpu.VMEM((GT,), jnp.float32),
        pltpu.VMEM((HS, CH), jnp.float32),
        pltpu.VMEM((HS, CH), jnp.float32),
        pltpu.VMEM((HS, CH), jnp.int32),
        pltpu.VMEM((HS, CH), jnp.int32),
        pltpu.VMEM_SHARED((GS,), jnp.float32),
        pltpu.SemaphoreType.DMA,
        pltpu.SemaphoreType.DMA,
    ],
)
def _sc_pass(src_hbm, dst_hbm, g_hbm, z_hbm, outs_hbm, outt_hbm,
             srcv, dstv, gv, accv, rows0, rows1, didx0, didx1, acc_sh,
             sem0, sem1):
    c = lax.axis_index("c")
    s = lax.axis_index("s")
    b = c * NS + s
    bufs = ((rows0, didx0, sem0), (rows1, didx1, sem1))

    pltpu.sync_copy(z_hbm.at[pl.ds(s * GSL, GSL)], acc_sh.at[pl.ds(s * GSL, GSL)])
    pltpu.sync_copy(z_hbm.at[pl.ds(0, GT)], accv)
    pltpu.sync_copy(g_hbm, gv)
    pltpu.sync_copy(src_hbm.at[b], srcv)
    pltpu.sync_copy(dst_hbm.at[b], dstv)
    plsc.subcore_barrier()

    def gather_and_fire(j, p):
        rows, didx, sem = bufs[p]
        for t in range(CH // 16):
            s0 = srcv[j, pl.ds(t * 16, 16)]
            d0 = dstv[j, pl.ds(t * 16, 16)]
            for k in range(HS):
                rows[k, pl.ds(t * 16, 16)] = plsc.load_gather(gv, [s0 + k * NP])
                didx[k, pl.ds(t * 16, 16)] = d0 + k * NP
            for k in range(HS, H):
                vals = plsc.load_gather(gv, [s0 + k * NP])
                plsc.addupdate_scatter(accv, [d0 + (k - HS) * NP], vals)
        for k in range(HS):
            pltpu.async_copy(rows.at[k], acc_sh.at[didx.at[k]], sem, add=True)

    def drain(p):
        rows, _, sem = bufs[p]
        for k in range(HS):
            pltpu.make_async_copy(z_hbm.at[pl.ds(0, CH)], rows.at[k], sem).wait()

    gather_and_fire(0, 0)
    gather_and_fire(1, 1)

    def step(jj, carry):
        for p in range(2):
            drain(p)
            gather_and_fire(jj * 2 + 2 + p, p)
        return carry

    lax.fori_loop(0, (R - 2) // 2, step, 0)
    drain(0)
    drain(1)

    pltpu.sync_copy(accv, outt_hbm.at[b])
    plsc.subcore_barrier()
    pltpu.sync_copy(acc_sh.at[pl.ds(s * GSL, GSL)], outs_hbm.at[c, pl.ds(s * GSL, GSL)])


def _tc_prep_body(x_ref, w_ref, deg2_ref, g1_ref, dr_ref):
    deg = deg2_ref[0:1, :] + deg2_ref[1:2, :] + 1.0
    dinv = lax.rsqrt(deg)
    ht = lax.dot_general(w_ref[...], x_ref[...], (((0,), (1,)), ((), ())),
                         preferred_element_type=jnp.float32)
    g1_ref[...] = ht * dinv
    dr_ref[...] = jnp.broadcast_to(dinv, (H, NP))


def _acc_sum(asp_ref, atl_ref):
    low = asp_ref[0] + asp_ref[1]
    hi = atl_ref[0]
    for w in range(1, NW):
        hi = hi + atl_ref[w]
    return jnp.concatenate([low, hi], axis=0)


def _tc_mid_body(asp_ref, atl_ref, g1_ref, dr_ref, b1_ref, g2_ref):
    a = _acc_sum(asp_ref, atl_ref) + g1_ref[...]
    out1 = dr_ref[...] * a + b1_ref[...]
    r = jnp.maximum(out1, 0.0)
    col = lax.broadcasted_iota(jnp.int32, (H, NP), 1)
    r = jnp.where(col < N, r, 0.0)
    g2_ref[...] = dr_ref[...] * r


def _tc_final_body(asp_ref, atl_ref, g2_ref, dr_ref, w2_ref, b2_ref, o_ref):
    u = dr_ref[...] * (_acc_sum(asp_ref, atl_ref) + g2_ref[...])
    logits = lax.dot_general(u, w2_ref[...], (((0,), (0,)), ((), ())),
                             preferred_element_type=jnp.float32) + b2_ref[...]
    m = jnp.max(logits, axis=1, keepdims=True)
    lse = jnp.log(jnp.sum(jnp.exp(logits - m), axis=1, keepdims=True))
    o_ref[...] = logits - m - lse


_tc_prep = pl.pallas_call(
    _tc_prep_body,
    out_shape=[
        jax.ShapeDtypeStruct((H, NP), jnp.float32),
        jax.ShapeDtypeStruct((H, NP), jnp.float32),
    ],
)

_tc_mid = pl.pallas_call(
    _tc_mid_body,
    out_shape=jax.ShapeDtypeStruct((H, NP), jnp.float32),
)

_tc_final = pl.pallas_call(
    _tc_final_body,
    out_shape=jax.ShapeDtypeStruct((NP, CLS), jnp.float32),
)


def kernel(x, edge_index, W1, b1, W2, b2):
    ei = edge_index.astype(jnp.int32)

    padd = jnp.full((EPAD - E,), N, jnp.int32)
    srcd = jnp.concatenate([ei[0], padd]).reshape(NW, R, CH)
    dstd = jnp.concatenate([ei[1], padd]).reshape(NW, R, CH)

    xp = jnp.pad(x.astype(jnp.float32), ((0, NP - N), (0, 0)))
    w1 = W1.astype(jnp.float32)
    b1c = b1.astype(jnp.float32).reshape(H, 1)
    w2 = W2.astype(jnp.float32)
    b2r = b2.astype(jnp.float32).reshape(1, CLS)
    z1 = jnp.zeros((NP,), jnp.float32)
    z5 = jnp.zeros((GF,), jnp.float32)

    degp = _sc_deg(dstd, z1)
    g1, dr = _tc_prep(xp, w1, degp)
    a1s, a1t = _sc_pass(srcd, dstd, g1.reshape(-1), z5)
    g2 = _tc_mid(a1s.reshape(NC, HS, NP), a1t.reshape(NW, HT, NP),
                 g1, dr, b1c)
    a2s, a2t = _sc_pass(srcd, dstd, g2.reshape(-1), z5)
    out = _tc_final(a2s.reshape(NC, HS, NP), a2t.reshape(NW, HT, NP),
                    g2, dr, w2, b2r)
    return out[:N]
